# speedup vs baseline: 3.2681x; 3.2681x over previous
import functools

import jax
import jax.numpy as jnp
from jax.experimental import pallas as pl

EPS = 1e-12


def _ln_kernel(x_ref, pe_ref, g_ref, b_ref, o_ref):
    xb = x_ref[0] + pe_ref[...]
    u = jnp.mean(xb, axis=-1, keepdims=True)
    d = xb - u
    s = jnp.mean(d * d, axis=-1, keepdims=True)
    o_ref[0] = g_ref[...] * (d * jax.lax.rsqrt(s + EPS)) + b_ref[...]


@functools.partial(jax.jit, static_argnames=("s_blk",))
def _run(x, pos_embed, gamma, beta, s_blk=512):
    B, S, D = x.shape
    gamma2 = gamma.reshape(1, D)
    beta2 = beta.reshape(1, D)
    grid = (B, S // s_blk)
    return pl.pallas_call(
        _ln_kernel,
        grid=grid,
        in_specs=[
            pl.BlockSpec((1, s_blk, D), lambda b, s: (b, s, 0)),
            pl.BlockSpec((s_blk, D), lambda b, s: (s, 0)),
            pl.BlockSpec((1, D), lambda b, s: (0, 0)),
            pl.BlockSpec((1, D), lambda b, s: (0, 0)),
        ],
        out_specs=pl.BlockSpec((1, s_blk, D), lambda b, s: (b, s, 0)),
        out_shape=jax.ShapeDtypeStruct((B, S, D), x.dtype),
    )(x, pos_embed, gamma2, beta2)


def kernel(x, pos_embed, gamma, beta):
    S = x.shape[1]
    return _run(x, pos_embed[:S], gamma, beta)


# speedup vs baseline: 3.5150x; 1.0756x over previous
import functools

import jax
import jax.numpy as jnp
from jax.experimental import pallas as pl

EPS = 1e-12


def _ln_kernel(x_ref, pe_ref, g_ref, b_ref, o_ref):
    xb = x_ref[0] + pe_ref[...]
    u = jnp.mean(xb, axis=-1, keepdims=True)
    d = xb - u
    s = jnp.mean(d * d, axis=-1, keepdims=True)
    o_ref[0] = g_ref[...] * (d * jax.lax.rsqrt(s + EPS)) + b_ref[...]


@functools.partial(jax.jit, static_argnames=("s_blk",))
def _run(x, pos_embed, gamma, beta, s_blk=512):
    B, S, D = x.shape
    gamma2 = gamma.reshape(1, D)
    beta2 = beta.reshape(1, D)
    grid = (S // s_blk, B)
    return pl.pallas_call(
        _ln_kernel,
        grid=grid,
        in_specs=[
            pl.BlockSpec((1, s_blk, D), lambda s, b: (b, s, 0)),
            pl.BlockSpec((s_blk, D), lambda s, b: (s, 0)),
            pl.BlockSpec((1, D), lambda s, b: (0, 0)),
            pl.BlockSpec((1, D), lambda s, b: (0, 0)),
        ],
        out_specs=pl.BlockSpec((1, s_blk, D), lambda s, b: (b, s, 0)),
        out_shape=jax.ShapeDtypeStruct((B, S, D), x.dtype),
    )(x, pos_embed, gamma2, beta2)


def kernel(x, pos_embed, gamma, beta):
    S = x.shape[1]
    return _run(x, pos_embed[:S], gamma, beta)


# speedup vs baseline: 4.4885x; 1.2769x over previous
import functools

import jax
import jax.numpy as jnp
from jax.experimental import pallas as pl

EPS = 1e-12


def _ln_kernel(x_ref, pe_ref, g_ref, b_ref, o_ref):
    xb = x_ref[...] + pe_ref[...][None, :, :]
    u = jnp.mean(xb, axis=-1, keepdims=True)
    d = xb - u
    s = jnp.mean(d * d, axis=-1, keepdims=True)
    o_ref[...] = g_ref[...] * (d * jax.lax.rsqrt(s + EPS)) + b_ref[...]


@functools.partial(jax.jit, static_argnames=("s_blk",))
def _run(x, pos_embed, gamma, beta, s_blk=512):
    B, S, D = x.shape
    gamma2 = gamma.reshape(1, D)
    beta2 = beta.reshape(1, D)
    grid = (S // s_blk,)
    return pl.pallas_call(
        _ln_kernel,
        grid=grid,
        in_specs=[
            pl.BlockSpec((B, s_blk, D), lambda s: (0, s, 0)),
            pl.BlockSpec((s_blk, D), lambda s: (s, 0)),
            pl.BlockSpec((1, D), lambda s: (0, 0)),
            pl.BlockSpec((1, D), lambda s: (0, 0)),
        ],
        out_specs=pl.BlockSpec((B, s_blk, D), lambda s: (0, s, 0)),
        out_shape=jax.ShapeDtypeStruct((B, S, D), x.dtype),
    )(x, pos_embed, gamma2, beta2)


def kernel(x, pos_embed, gamma, beta):
    S = x.shape[1]
    return _run(x, pos_embed[:S], gamma, beta)
